# baseline (speedup 1.0000x reference)
import functools

import jax
import jax.numpy as jnp
import numpy as np
from jax import lax
from jax.experimental import pallas as pl
from jax.experimental.pallas import tpu as pltpu
from jax.experimental.pallas import tpu_sc as plsc

B = 8
C, H, W = 96, 224, 224
N = C * H * W
K = N // 2
NC, NS = 2, 16
NW = NC * NS
CPT = C // NW
HR = H // 2
NCHK = CPT * 2
VROW = W // 16

PASS_BITS = (11, 11, 10)
PASS_SHIFT = (21, 10, 0)

_MESH = plsc.VectorSubcoreMesh(core_axis_name="c", subcore_axis_name="s")


def _make_hist_pass(pass_idx: int):
    nbits = PASS_BITS[pass_idx]
    shift = PASS_SHIFT[pass_idx]
    nb = 1 << nbits
    have_prefix = pass_idx > 0
    pshift = shift + nbits

    mask_c = nb - 1
    c0 = (0x80000000 >> shift) & mask_c
    mc = mask_c ^ c0
    if have_prefix:
        tbp = 1 << (31 - pshift)
        m1p = (1 << (32 - pshift)) - 1
        mp = m1p ^ tbp

    @functools.partial(
        pl.kernel,
        out_type=jax.ShapeDtypeStruct((NW * B * nb,), jnp.int32),
        mesh=_MESH,
        scratch_types=[
            pltpu.VMEM((2, HR, W), jnp.float32),
            pltpu.VMEM((16,), jnp.uint32),
            pltpu.VMEM((nb,), jnp.int32),
            pltpu.VMEM((nb * 16,), jnp.int32),
            pltpu.SemaphoreType.DMA((2,)),
        ],
        compiler_params=pltpu.CompilerParams(needs_layout_passes=False),
    )
    def hist_kernel(x_hbm, pref_hbm, out_hbm, buf, pv, rh, hist, sem):
        wid = lax.axis_index("c") * NS + lax.axis_index("s")
        lanes = lax.iota(jnp.int32, 16)
        lanes_u = lax.iota(jnp.uint32, 16)
        cav = (lanes_u * jnp.uint32(nb)) ^ jnp.uint32(c0)
        ones = jnp.full((16,), 1, jnp.int32)
        zv = jnp.zeros((16,), jnp.int32)
        ch0 = wid * CPT

        def issue(row, k):
            slot = k & 1
            pltpu.async_copy(
                x_hbm.at[row, ch0 + (k >> 1), pl.ds((k & 1) * HR, HR)],
                buf.at[slot], sem.at[slot])

        issue(0, 0)

        def zero_body(i, _):
            hist[pl.ds(i * 16, 16)] = zv
            return 0
        lax.fori_loop(0, nb, zero_body, 0)

        def row_body(row, _):
            if have_prefix:
                pltpu.sync_copy(pref_hbm.at[pl.ds(row * 16, 16)], pv)
                pvv2 = pv[...] ^ jnp.uint32(tbp)

            def chunk_body(k, _):
                slot = k & 1
                pltpu.make_async_copy(
                    x_hbm.at[0, 0, pl.ds(0, HR)], buf.at[slot], sem.at[slot]
                ).wait()

                @pl.when(k + 1 < NCHK)
                def _():
                    issue(row, k + 1)

                @pl.when((k + 1 >= NCHK) & (row + 1 < B))
                def _():
                    issue(row + 1, 0)

                @plsc.parallel_loop(0, HR)
                def vec_body(r):
                    xs = [buf[slot, r, pl.ds(vv * 16, 16)] for vv in range(VROW)]
                    us = [lax.bitcast_convert_type(xv, jnp.uint32) for xv in xs]
                    ss = [lax.bitcast_convert_type(
                        lax.shift_right_arithmetic(
                            lax.bitcast_convert_type(u, jnp.int32), 31),
                        jnp.uint32) for u in us]
                    addrs = []
                    for u, s in zip(us, ss):
                        q = lax.shift_right_logical(u, jnp.uint32(shift)) \
                            if shift else u
                        if shift + nbits < 32:
                            q = q & jnp.uint32(mask_c)
                        addrs.append(lax.bitcast_convert_type(
                            q ^ (s & jnp.uint32(mc)) ^ cav, jnp.int32))
                    if have_prefix:
                        acts = [
                            (lax.shift_right_logical(u, jnp.uint32(pshift))
                             ^ (s & jnp.uint32(mp))) == pvv2
                            for u, s in zip(us, ss)
                        ]
                        for addr, act in zip(addrs, acts):
                            plsc.addupdate_scatter(hist, [addr], ones, mask=act)
                    else:
                        for addr in addrs:
                            plsc.addupdate_scatter(hist, [addr], ones)

                return 0

            lax.fori_loop(0, NCHK, chunk_body, 0)

            def red_body(g, _):
                acc = zv
                for l in range(16):
                    off = l * nb + g * 16
                    acc = acc + hist[pl.ds(off, 16)]
                    hist[pl.ds(off, 16)] = zv
                rh[pl.ds(g * 16, 16)] = acc
                return 0

            lax.fori_loop(0, nb // 16, red_body, 0)
            pltpu.sync_copy(rh, out_hbm.at[pl.ds((wid * B + row) * nb, nb)])
            return 0

        lax.fori_loop(0, B, row_body, 0)

    return hist_kernel, nb


_HIST_PASSES = tuple(_make_hist_pass(i) for i in range(len(PASS_BITS)))

CB = 8
NG = C // CB


def _mask_body(thr_ref, x_ref, o_ref):
    t = thr_ref[0, 0, 0, 0]
    xb = x_ref[...]
    o_ref[...] = jnp.where(xb >= t, xb, jnp.float32(0.0))


_mask_call = pl.pallas_call(
    _mask_body,
    grid=(B, NG),
    in_specs=[
        pl.BlockSpec((1, 1, 1, 1), lambda i, j: (i, 0, 0, 0),
                     memory_space=pltpu.SMEM),
        pl.BlockSpec((1, CB, H, W), lambda i, j: (i, j, 0, 0)),
    ],
    out_specs=pl.BlockSpec((1, CB, H, W), lambda i, j: (i, j, 0, 0)),
    out_shape=jax.ShapeDtypeStruct((B, C, H, W), jnp.float32),
    compiler_params=pltpu.CompilerParams(
        dimension_semantics=("parallel", "parallel")),
)


def _select_bin(hist_rows, kk, nb):
    rc = jnp.cumsum(hist_rows[:, ::-1], axis=1)[:, ::-1]
    ge = rc >= kk[:, None]
    binidx = jnp.argmax(jnp.where(ge, jnp.arange(nb)[None, :], -1), axis=1)
    rc_b = jnp.take_along_axis(rc, binidx[:, None], axis=1)[:, 0]
    h_b = jnp.take_along_axis(hist_rows, binidx[:, None], axis=1)[:, 0]
    k_next = kk - (rc_b - h_b)
    return binidx, k_next


@jax.jit
def kernel(x):
    prefix = jnp.zeros((B,), jnp.uint32)
    kk = jnp.full((B,), K, jnp.int32)

    for pidx, (hist_pass, nb) in enumerate(_HIST_PASSES):
        pref_splat = jnp.broadcast_to(prefix[:, None], (B, 16)).reshape(-1)
        hist_flat = hist_pass(x, pref_splat)
        hist_rows = hist_flat.reshape(NW, B, nb).sum(axis=0)
        binidx, kk = _select_bin(hist_rows, kk, nb)
        prefix = (prefix << PASS_BITS[pidx]) | binidx.astype(jnp.uint32)

    sgn = prefix >> 31
    u = jnp.where(sgn == 1, prefix ^ jnp.uint32(0x80000000), ~prefix)
    topval = lax.bitcast_convert_type(u, jnp.float32)

    return _mask_call(topval[:, None, None, None], x)


# speedup vs baseline: 1.0515x; 1.0515x over previous
import functools

import jax
import jax.numpy as jnp
import numpy as np
from jax import lax
from jax.experimental import pallas as pl
from jax.experimental.pallas import tpu as pltpu
from jax.experimental.pallas import tpu_sc as plsc

B = 8
C, H, W = 96, 224, 224
N = C * H * W
K = N // 2
NC, NS = 2, 16
NW = NC * NS
CPT = C // NW
HR = H // 2
NCHK = CPT * 2
VROW = W // 16

PASS_BITS = (11, 11, 10)
PASS_SHIFT = (21, 10, 0)

_MESH = plsc.VectorSubcoreMesh(core_axis_name="c", subcore_axis_name="s")


def _make_hist_pass(pass_idx: int):
    nbits = PASS_BITS[pass_idx]
    shift = PASS_SHIFT[pass_idx]
    nb = 1 << nbits
    have_prefix = pass_idx > 0
    pshift = shift + nbits

    @functools.partial(
        pl.kernel,
        out_type=jax.ShapeDtypeStruct((NW * B * nb,), jnp.int32),
        mesh=_MESH,
        scratch_types=[
            pltpu.VMEM((2, HR, W), jnp.float32),
            pltpu.VMEM((16,), jnp.uint32),
            pltpu.VMEM((nb,), jnp.int32),
            pltpu.VMEM((nb * 16,), jnp.int32),
            pltpu.SemaphoreType.DMA((2,)),
        ],
        compiler_params=pltpu.CompilerParams(
            needs_layout_passes=False, disable_bounds_checks=True),
    )
    def hist_kernel(x_hbm, pref_hbm, out_hbm, buf, pv, rh, hist, sem):
        wid = lax.axis_index("c") * NS + lax.axis_index("s")
        lanes = lax.iota(jnp.int32, 16)
        lvec = lanes * nb
        ones = jnp.full((16,), 1, jnp.int32)
        zv = jnp.zeros((16,), jnp.int32)
        ch0 = wid * CPT

        def issue(row, k):
            slot = k & 1
            pltpu.async_copy(
                x_hbm.at[row, ch0 + (k >> 1), pl.ds((k & 1) * HR, HR)],
                buf.at[slot], sem.at[slot])

        issue(0, 0)

        def zero_body(i, _):
            hist[pl.ds(i * 16, 16)] = zv
            return 0
        lax.fori_loop(0, nb, zero_body, 0)

        def row_body(row, _):
            if have_prefix:
                pltpu.sync_copy(pref_hbm.at[pl.ds(row * 16, 16)], pv)
                pvv = pv[...]

            def chunk_body(k, _):
                slot = k & 1
                pltpu.make_async_copy(
                    x_hbm.at[0, 0, pl.ds(0, HR)], buf.at[slot], sem.at[slot]
                ).wait()

                @pl.when(k + 1 < NCHK)
                def _():
                    issue(row, k + 1)

                @pl.when((k + 1 >= NCHK) & (row + 1 < B))
                def _():
                    issue(row + 1, 0)

                @plsc.parallel_loop(0, HR)
                def vec_body(r):
                    xs = [buf[slot, r, pl.ds(vv * 16, 16)] for vv in range(VROW)]
                    ms = []
                    for xv in xs:
                        u = lax.bitcast_convert_type(xv, jnp.uint32)
                        s = lax.shift_right_logical(u, jnp.uint32(31))
                        ms.append(u ^ ((jnp.uint32(0) - s) | jnp.uint32(0x80000000)))
                    addrs = [
                        lvec + lax.convert_element_type(
                            lax.shift_right_logical(m, jnp.uint32(shift))
                            & jnp.uint32(nb - 1), jnp.int32)
                        for m in ms
                    ]
                    if have_prefix:
                        acts = [lax.shift_right_logical(m, jnp.uint32(pshift)) == pvv
                                for m in ms]
                        for addr, act in zip(addrs, acts):
                            plsc.addupdate_scatter(hist, [addr], ones, mask=act)
                    else:
                        for addr in addrs:
                            plsc.addupdate_scatter(hist, [addr], ones)

                return 0

            lax.fori_loop(0, NCHK, chunk_body, 0)

            def red_body(g, _):
                acc = zv
                for l in range(16):
                    off = l * nb + g * 16
                    acc = acc + hist[pl.ds(off, 16)]
                    hist[pl.ds(off, 16)] = zv
                rh[pl.ds(g * 16, 16)] = acc
                return 0

            lax.fori_loop(0, nb // 16, red_body, 0)
            pltpu.sync_copy(rh, out_hbm.at[pl.ds((wid * B + row) * nb, nb)])
            return 0

        lax.fori_loop(0, B, row_body, 0)

    return hist_kernel, nb


_HIST_PASSES = tuple(_make_hist_pass(i) for i in range(len(PASS_BITS)))

CB = 8
NG = C // CB


def _mask_body(thr_ref, x_ref, o_ref):
    t = thr_ref[0, 0, 0, 0]
    xb = x_ref[...]
    o_ref[...] = jnp.where(xb >= t, xb, jnp.float32(0.0))


_mask_call = pl.pallas_call(
    _mask_body,
    grid=(B, NG),
    in_specs=[
        pl.BlockSpec((1, 1, 1, 1), lambda i, j: (i, 0, 0, 0),
                     memory_space=pltpu.SMEM),
        pl.BlockSpec((1, CB, H, W), lambda i, j: (i, j, 0, 0)),
    ],
    out_specs=pl.BlockSpec((1, CB, H, W), lambda i, j: (i, j, 0, 0)),
    out_shape=jax.ShapeDtypeStruct((B, C, H, W), jnp.float32),
    compiler_params=pltpu.CompilerParams(
        dimension_semantics=("parallel", "parallel")),
)


def _select_bin(hist_rows, kk, nb):
    rc = jnp.cumsum(hist_rows[:, ::-1], axis=1)[:, ::-1]
    ge = rc >= kk[:, None]
    binidx = jnp.argmax(jnp.where(ge, jnp.arange(nb)[None, :], -1), axis=1)
    rc_b = jnp.take_along_axis(rc, binidx[:, None], axis=1)[:, 0]
    h_b = jnp.take_along_axis(hist_rows, binidx[:, None], axis=1)[:, 0]
    k_next = kk - (rc_b - h_b)
    return binidx, k_next


@jax.jit
def kernel(x):
    prefix = jnp.zeros((B,), jnp.uint32)
    kk = jnp.full((B,), K, jnp.int32)

    for pidx, (hist_pass, nb) in enumerate(_HIST_PASSES):
        pref_splat = jnp.broadcast_to(prefix[:, None], (B, 16)).reshape(-1)
        hist_flat = hist_pass(x, pref_splat)
        hist_rows = hist_flat.reshape(NW, B, nb).sum(axis=0)
        binidx, kk = _select_bin(hist_rows, kk, nb)
        prefix = (prefix << PASS_BITS[pidx]) | binidx.astype(jnp.uint32)

    sgn = prefix >> 31
    u = jnp.where(sgn == 1, prefix ^ jnp.uint32(0x80000000), ~prefix)
    topval = lax.bitcast_convert_type(u, jnp.float32)

    return _mask_call(topval[:, None, None, None], x)


# speedup vs baseline: 1.0519x; 1.0003x over previous
import functools

import jax
import jax.numpy as jnp
import numpy as np
from jax import lax
from jax.experimental import pallas as pl
from jax.experimental.pallas import tpu as pltpu
from jax.experimental.pallas import tpu_sc as plsc

B = 8
C, H, W = 96, 224, 224
N = C * H * W
K = N // 2
NC, NS = 2, 16
NW = NC * NS
CPT = C // NW
HR = H // 2
NCHK = CPT * 2
VROW = W // 16

PASS_BITS = (11, 11, 10)
PASS_SHIFT = (21, 10, 0)

_MESH = plsc.VectorSubcoreMesh(core_axis_name="c", subcore_axis_name="s")


def _make_hist_pass(pass_idx: int):
    nbits = PASS_BITS[pass_idx]
    shift = PASS_SHIFT[pass_idx]
    nb = 1 << nbits
    have_prefix = pass_idx > 0
    pshift = shift + nbits

    @functools.partial(
        pl.kernel,
        out_type=jax.ShapeDtypeStruct((NW * B * nb,), jnp.int32),
        mesh=_MESH,
        scratch_types=[
            pltpu.VMEM((2, HR, W), jnp.float32),
            pltpu.VMEM((16,), jnp.uint32),
            pltpu.VMEM((nb,), jnp.int32),
            pltpu.VMEM((nb * 16,), jnp.int32),
            pltpu.SemaphoreType.DMA((2,)),
        ],
        compiler_params=pltpu.CompilerParams(
            needs_layout_passes=False, disable_bounds_checks=True),
    )
    def hist_kernel(x_hbm, pref_hbm, out_hbm, buf, pv, rh, hist, sem):
        wid = lax.axis_index("c") * NS + lax.axis_index("s")
        lanes = lax.iota(jnp.int32, 16)
        lvec = lanes * nb
        ones = jnp.full((16,), 1, jnp.int32)
        zv = jnp.zeros((16,), jnp.int32)
        ch0 = wid * CPT

        def issue(row, k):
            slot = k & 1
            pltpu.async_copy(
                x_hbm.at[row, ch0 + (k >> 1), pl.ds((k & 1) * HR, HR)],
                buf.at[slot], sem.at[slot])

        issue(0, 0)

        def zero_body(i, _):
            hist[pl.ds(i * 16, 16)] = zv
            return 0
        lax.fori_loop(0, nb, zero_body, 0)

        def row_body(row, _):
            if have_prefix:
                pltpu.sync_copy(pref_hbm.at[pl.ds(row * 16, 16)], pv)
                pvv = pv[...]

            def chunk_body(k, _):
                slot = k & 1
                pltpu.make_async_copy(
                    x_hbm.at[0, 0, pl.ds(0, HR)], buf.at[slot], sem.at[slot]
                ).wait()

                @pl.when(k + 1 < NCHK)
                def _():
                    issue(row, k + 1)

                @pl.when((k + 1 >= NCHK) & (row + 1 < B))
                def _():
                    issue(row + 1, 0)

                @plsc.parallel_loop(0, HR)
                def vec_body(r):
                    xs = [buf[slot, r, pl.ds(vv * 16, 16)] for vv in range(VROW)]
                    ms = []
                    for xv in xs:
                        u = lax.bitcast_convert_type(xv, jnp.uint32)
                        s = lax.shift_right_logical(u, jnp.uint32(31))
                        ms.append(u ^ ((jnp.uint32(0) - s) | jnp.uint32(0x80000000)))
                    def binof(m):
                        if shift + nbits == 32:
                            return lax.shift_right_logical(m, jnp.uint32(shift))
                        hi = lax.shift_left(m, jnp.uint32(32 - shift - nbits))
                        return lax.shift_right_logical(hi, jnp.uint32(32 - nbits))

                    addrs = [
                        lvec | lax.bitcast_convert_type(binof(m), jnp.int32)
                        for m in ms
                    ]
                    if have_prefix:
                        acts = [lax.shift_right_logical(m, jnp.uint32(pshift)) == pvv
                                for m in ms]
                        for addr, act in zip(addrs, acts):
                            plsc.addupdate_scatter(hist, [addr], ones, mask=act)
                    else:
                        for addr in addrs:
                            plsc.addupdate_scatter(hist, [addr], ones)

                return 0

            lax.fori_loop(0, NCHK, chunk_body, 0)

            def red_body(g, _):
                acc = zv
                for l in range(16):
                    off = l * nb + g * 16
                    acc = acc + hist[pl.ds(off, 16)]
                    hist[pl.ds(off, 16)] = zv
                rh[pl.ds(g * 16, 16)] = acc
                return 0

            lax.fori_loop(0, nb // 16, red_body, 0)
            pltpu.sync_copy(rh, out_hbm.at[pl.ds((wid * B + row) * nb, nb)])
            return 0

        lax.fori_loop(0, B, row_body, 0)

    return hist_kernel, nb


_HIST_PASSES = tuple(_make_hist_pass(i) for i in range(len(PASS_BITS)))

CB = 8
NG = C // CB


def _mask_body(thr_ref, x_ref, o_ref):
    t = thr_ref[0, 0, 0, 0]
    xb = x_ref[...]
    o_ref[...] = jnp.where(xb >= t, xb, jnp.float32(0.0))


_mask_call = pl.pallas_call(
    _mask_body,
    grid=(B, NG),
    in_specs=[
        pl.BlockSpec((1, 1, 1, 1), lambda i, j: (i, 0, 0, 0),
                     memory_space=pltpu.SMEM),
        pl.BlockSpec((1, CB, H, W), lambda i, j: (i, j, 0, 0)),
    ],
    out_specs=pl.BlockSpec((1, CB, H, W), lambda i, j: (i, j, 0, 0)),
    out_shape=jax.ShapeDtypeStruct((B, C, H, W), jnp.float32),
    compiler_params=pltpu.CompilerParams(
        dimension_semantics=("parallel", "parallel")),
)


def _select_bin(hist_rows, kk, nb):
    rc = jnp.cumsum(hist_rows[:, ::-1], axis=1)[:, ::-1]
    ge = rc >= kk[:, None]
    binidx = jnp.argmax(jnp.where(ge, jnp.arange(nb)[None, :], -1), axis=1)
    rc_b = jnp.take_along_axis(rc, binidx[:, None], axis=1)[:, 0]
    h_b = jnp.take_along_axis(hist_rows, binidx[:, None], axis=1)[:, 0]
    k_next = kk - (rc_b - h_b)
    return binidx, k_next


@jax.jit
def kernel(x):
    prefix = jnp.zeros((B,), jnp.uint32)
    kk = jnp.full((B,), K, jnp.int32)

    for pidx, (hist_pass, nb) in enumerate(_HIST_PASSES):
        pref_splat = jnp.broadcast_to(prefix[:, None], (B, 16)).reshape(-1)
        hist_flat = hist_pass(x, pref_splat)
        hist_rows = hist_flat.reshape(NW, B, nb).sum(axis=0)
        binidx, kk = _select_bin(hist_rows, kk, nb)
        prefix = (prefix << PASS_BITS[pidx]) | binidx.astype(jnp.uint32)

    sgn = prefix >> 31
    u = jnp.where(sgn == 1, prefix ^ jnp.uint32(0x80000000), ~prefix)
    topval = lax.bitcast_convert_type(u, jnp.float32)

    return _mask_call(topval[:, None, None, None], x)
